# baseline (speedup 1.0000x reference)
import jax
import jax.numpy as jnp
from jax import lax
from jax.experimental import pallas as pl
from jax.experimental.pallas import tpu as pltpu
from jax.experimental.pallas import tpu_sc as plsc

_N = 10000
_G = 64
_NT = 16
_FULL = 640
_LAST = _N - 15 * _FULL
_CH = 80


def _fetch_and_count(x_hbm, b_hbm, colbuf, bbuf2d, idx2d, sem, cnts,
                     lane, base, n_rows):
    n_chunks = n_rows // _CH
    for c in range(n_chunks):
        pltpu.sync_copy(b_hbm.at[pl.ds(base + c * _CH, _CH)], bbuf2d.at[c])
    for c in range(n_chunks):
        for j in range(_CH // 16):
            idx2d[c, pl.ds(j * 16, 16)] = (base + c * _CH + j * 16 + lane) * 128
    copies = [
        pltpu.async_copy(x_hbm.at[idx2d.at[c]],
                         colbuf.at[pl.ds(c * _CH, _CH)], sem)
        for c in range(n_chunks)
    ]
    cc = (lane + 1).astype(jnp.float32)
    for c in range(n_chunks):
        c_splat = jnp.full((16,), c, jnp.int32)
        for j in range(_CH // 16):
            col = j * 16 + lane
            b = bbuf2d[c, pl.ds(j * 16, 16)]
            bn = plsc.load_gather(bbuf2d, [c_splat, jnp.minimum(col + 1, _CH - 1)])
            m_end = (lane == 15) | (b != bn)
            m_int = (lane < 15) & (b != bn)
            plsc.addupdate_scatter(cnts, [b], cc, mask=m_end)
            plsc.addupdate_scatter(cnts, [bn], -cc, mask=m_int)
    for d in copies:
        d.wait()


def _scatter_sums(colbuf, bbuf2d, sem, sh_s, n_rows):
    copies = [
        pltpu.async_copy(colbuf.at[pl.ds(c * _CH, _CH)],
                         sh_s.at[bbuf2d.at[c]], sem, add=True)
        for c in range(n_rows // _CH)
    ]
    for d in copies:
        d.wait()


def _body(x_hbm, b_hbm, out_hbm, colbuf, bbuf2d, idx2d, idq, sem, cnts,
          sh_s, sh_c, t_s, t_c, obuf):
    wid = lax.axis_index("s")
    lane = lax.iota(jnp.int32, 16)
    zeros16 = jnp.zeros((16,), jnp.float32)

    @pl.when(wid == 0)
    def _():
        for j in range(_G // 16):
            obuf[pl.ds(j * 16, 16)] = zeros16
        pltpu.sync_copy(obuf, sh_s)
        pltpu.sync_copy(obuf, sh_c)

    for j in range(_G // 16):
        cnts[pl.ds(j * 16, 16)] = zeros16
        idq[pl.ds(j * 16, 16)] = j * 16 + lane

    @pl.when(wid < _NT - 1)
    def _():
        _fetch_and_count(x_hbm, b_hbm, colbuf, bbuf2d, idx2d, sem, cnts,
                         lane, wid * _FULL, _FULL)

    @pl.when(wid == _NT - 1)
    def _():
        _fetch_and_count(x_hbm, b_hbm, colbuf, bbuf2d, idx2d, sem, cnts,
                         lane, (_NT - 1) * _FULL, _LAST)

    plsc.subcore_barrier()

    @pl.when(wid < _NT - 1)
    def _():
        _scatter_sums(colbuf, bbuf2d, sem, sh_s, _FULL)

    @pl.when(wid == _NT - 1)
    def _():
        _scatter_sums(colbuf, bbuf2d, sem, sh_s, _LAST)

    pltpu.sync_copy(cnts, sh_c.at[idq], add=True)
    plsc.subcore_barrier()

    @pl.when(wid == 0)
    def _():
        pltpu.sync_copy(sh_s, t_s)
        pltpu.sync_copy(sh_c, t_c)
        for j in range(_G // 16):
            obuf[pl.ds(j * 16, 16)] = (t_s[pl.ds(j * 16, 16)]
                                       / t_c[pl.ds(j * 16, 16)])
        pltpu.sync_copy(obuf, out_hbm)


@jax.jit
def _seg_mean(x, batch):
    mesh = plsc.VectorSubcoreMesh(
        core_axis_name="c", subcore_axis_name="s", num_cores=1)
    f = pl.kernel(
        _body,
        out_type=jax.ShapeDtypeStruct((_G,), jnp.float32),
        mesh=mesh,
        compiler_params=pltpu.CompilerParams(needs_layout_passes=False),
        scratch_types=[
            pltpu.VMEM((_FULL,), jnp.float32),
            pltpu.VMEM((_FULL // _CH, _CH), jnp.int32),
            pltpu.VMEM((_FULL // _CH, _CH), jnp.int32),
            pltpu.VMEM((_G,), jnp.int32),
            pltpu.SemaphoreType.DMA,
            pltpu.VMEM((_G,), jnp.float32),
            pltpu.VMEM_SHARED((_G,), jnp.float32),
            pltpu.VMEM_SHARED((_G,), jnp.float32),
            pltpu.VMEM((_G,), jnp.float32),
            pltpu.VMEM((_G,), jnp.float32),
            pltpu.VMEM((_G,), jnp.float32),
        ],
    )
    return f(x.reshape(-1), batch)


def kernel(x, edge_index, edge_attr, batch):
    out = _seg_mean(x, batch.astype(jnp.int32))
    return out[:, None]


# speedup vs baseline: 1.0920x; 1.0920x over previous
import jax
import jax.numpy as jnp
from jax import lax
from jax.experimental import pallas as pl
from jax.experimental.pallas import tpu as pltpu
from jax.experimental.pallas import tpu_sc as plsc

_N = 10000
_G = 64
_NT = 16
_FULL = 640
_LAST = _N - 15 * _FULL


def _count_pass(bbuf, cnts, lane, n_rows):
    cc = (lane + 1).astype(jnp.float32)
    for it in range(n_rows // 16):
        off = it * 16
        b = bbuf[pl.ds(off, 16)]
        bn = bbuf[pl.ds(off + 1, 16)]
        m_end = (lane == 15) | (b != bn)
        m_int = (lane < 15) & (b != bn)
        plsc.addupdate_scatter(cnts, [b], cc, mask=m_end)
        plsc.addupdate_scatter(cnts, [bn], -cc, mask=m_int)


def _sum_pass(colbuf, bbuf, sums, lane, n_rows):
    for it in range(n_rows // 16):
        off = it * 16
        b = bbuf[pl.ds(off, 16)]
        bn = bbuf[pl.ds(off + 1, 16)]
        c = jnp.cumsum(colbuf[pl.ds(off, 16)])
        m_end = (lane == 15) | (b != bn)
        m_int = (lane < 15) & (b != bn)
        plsc.addupdate_scatter(sums, [b], c, mask=m_end)
        plsc.addupdate_scatter(sums, [bn], -c, mask=m_int)


def _body(x_hbm, b_hbm, out_hbm, colbuf, bbuf, idxb, sem, sums, cnts,
          sh_s, sh_c, t_s, t_c, obuf):
    wid = lax.axis_index("s")
    lane = lax.iota(jnp.int32, 16)
    zeros16 = jnp.zeros((16,), jnp.float32)
    for j in range(_G // 16):
        sums[pl.ds(j * 16, 16)] = zeros16
        cnts[pl.ds(j * 16, 16)] = zeros16

    base = wid * _FULL

    @pl.when(wid < _NT - 1)
    def _():
        pltpu.sync_copy(b_hbm.at[pl.ds(base, _FULL)], bbuf.at[pl.ds(0, _FULL)])

    @pl.when(wid == _NT - 1)
    def _():
        pltpu.sync_copy(b_hbm.at[pl.ds((_NT - 1) * _FULL, _LAST)],
                        bbuf.at[pl.ds(0, _LAST)])

    for it in range(_FULL // 16):
        idxb[pl.ds(it * 16, 16)] = jnp.minimum(base + it * 16 + lane,
                                               _N - 1) * 128
    gather = pltpu.async_copy(x_hbm.at[idxb], colbuf, sem)

    @pl.when(wid < _NT - 1)
    def _():
        _count_pass(bbuf, cnts, lane, _FULL)

    @pl.when(wid == _NT - 1)
    def _():
        _count_pass(bbuf, cnts, lane, _LAST)

    gather.wait()

    @pl.when(wid < _NT - 1)
    def _():
        _sum_pass(colbuf, bbuf, sums, lane, _FULL)

    @pl.when(wid == _NT - 1)
    def _():
        _sum_pass(colbuf, bbuf, sums, lane, _LAST)

    pltpu.sync_copy(sums, sh_s.at[pl.ds(wid * _G, _G)])
    pltpu.sync_copy(cnts, sh_c.at[pl.ds(wid * _G, _G)])
    plsc.subcore_barrier()

    @pl.when(wid == 0)
    def _():
        pltpu.sync_copy(sh_s, t_s)
        pltpu.sync_copy(sh_c, t_c)
        for j in range(_G // 16):
            acc_s = zeros16
            acc_c = zeros16
            for r in range(_NT):
                acc_s = acc_s + t_s[pl.ds(r * _G + j * 16, 16)]
                acc_c = acc_c + t_c[pl.ds(r * _G + j * 16, 16)]
            obuf[pl.ds(j * 16, 16)] = acc_s / acc_c
        pltpu.sync_copy(obuf, out_hbm)


@jax.jit
def _seg_mean(x, batch):
    mesh = plsc.VectorSubcoreMesh(
        core_axis_name="c", subcore_axis_name="s", num_cores=1)
    f = pl.kernel(
        _body,
        out_type=jax.ShapeDtypeStruct((_G,), jnp.float32),
        mesh=mesh,
        compiler_params=pltpu.CompilerParams(needs_layout_passes=False),
        scratch_types=[
            pltpu.VMEM((_FULL,), jnp.float32),
            pltpu.VMEM((_FULL + 16,), jnp.int32),
            pltpu.VMEM((_FULL,), jnp.int32),
            pltpu.SemaphoreType.DMA,
            pltpu.VMEM((_G,), jnp.float32),
            pltpu.VMEM((_G,), jnp.float32),
            pltpu.VMEM_SHARED((_NT * _G,), jnp.float32),
            pltpu.VMEM_SHARED((_NT * _G,), jnp.float32),
            pltpu.VMEM((_NT * _G,), jnp.float32),
            pltpu.VMEM((_NT * _G,), jnp.float32),
            pltpu.VMEM((_G,), jnp.float32),
        ],
    )
    return f(x.reshape(-1), batch)


def kernel(x, edge_index, edge_attr, batch):
    out = _seg_mean(x, batch.astype(jnp.int32))
    return out[:, None]


# speedup vs baseline: 1.2218x; 1.1188x over previous
import jax
import jax.numpy as jnp
from jax import lax
from jax.experimental import pallas as pl
from jax.experimental.pallas import tpu as pltpu
from jax.experimental.pallas import tpu_sc as plsc

_N = 10000
_G = 64
_NT = 16
_W = 640
_SH = 624


def _body(x_hbm, b_hbm, out_hbm, colbuf, bbuf, idxb, sem, sums, cnts,
          sh_s, sh_c, t_s, t_c, obuf):
    wid = lax.axis_index("s")
    lane = lax.iota(jnp.int32, 16)
    zeros16 = jnp.zeros((16,), jnp.float32)
    for j in range(_G // 16):
        sums[pl.ds(j * 16, 16)] = zeros16
        cnts[pl.ds(j * 16, 16)] = zeros16

    base = jnp.minimum(wid * _SH, _N - _W)
    nv = jnp.where(wid == _NT - 1, _W // 16, _SH // 16)

    pltpu.sync_copy(b_hbm.at[pl.ds(base, _W)], bbuf.at[pl.ds(0, _W)])

    def idx_body(it, carry):
        idxb[pl.ds(it * 16, 16)] = (base + it * 16 + lane) * 128
        return carry

    lax.fori_loop(0, _W // 16, idx_body, 0)
    gather = pltpu.async_copy(x_hbm.at[idxb], colbuf, sem)

    cc = (lane + 1).astype(jnp.float32)

    def count_body(it, carry):
        off = it * 16
        b = bbuf[pl.ds(off, 16)]
        bn = bbuf[pl.ds(off + 1, 16)]
        m_end = (lane == 15) | (b != bn)
        m_int = (lane < 15) & (b != bn)
        plsc.addupdate_scatter(cnts, [b], cc, mask=m_end)
        plsc.addupdate_scatter(cnts, [bn], -cc, mask=m_int)
        return carry

    lax.fori_loop(0, nv, count_body, 0)
    gather.wait()

    def sum_body(it, carry):
        off = it * 16
        b = bbuf[pl.ds(off, 16)]
        bn = bbuf[pl.ds(off + 1, 16)]
        c = jnp.cumsum(colbuf[pl.ds(off, 16)])
        m_end = (lane == 15) | (b != bn)
        m_int = (lane < 15) & (b != bn)
        plsc.addupdate_scatter(sums, [b], c, mask=m_end)
        plsc.addupdate_scatter(sums, [bn], -c, mask=m_int)
        return carry

    lax.fori_loop(0, nv, sum_body, 0)

    pltpu.sync_copy(sums, sh_s.at[pl.ds(wid * _G, _G)])
    pltpu.sync_copy(cnts, sh_c.at[pl.ds(wid * _G, _G)])
    plsc.subcore_barrier()

    @pl.when(wid == 0)
    def _():
        pltpu.sync_copy(sh_s, t_s)
        pltpu.sync_copy(sh_c, t_c)

        def red_body(r, accs):
            return tuple(
                accs[j] + t_s[pl.ds(r * _G + j * 16, 16)] if j < _G // 16
                else accs[j] + t_c[pl.ds(r * _G + (j - _G // 16) * 16, 16)]
                for j in range(2 * (_G // 16))
            )

        accs = lax.fori_loop(0, _NT, red_body, (zeros16,) * (2 * (_G // 16)))
        for j in range(_G // 16):
            obuf[pl.ds(j * 16, 16)] = accs[j] / accs[j + _G // 16]
        pltpu.sync_copy(obuf, out_hbm)


@jax.jit
def _seg_mean(x, batch):
    mesh = plsc.VectorSubcoreMesh(
        core_axis_name="c", subcore_axis_name="s", num_cores=1)
    f = pl.kernel(
        _body,
        out_type=jax.ShapeDtypeStruct((_G,), jnp.float32),
        mesh=mesh,
        compiler_params=pltpu.CompilerParams(needs_layout_passes=False),
        scratch_types=[
            pltpu.VMEM((_W,), jnp.float32),
            pltpu.VMEM((_W + 16,), jnp.int32),
            pltpu.VMEM((_W,), jnp.int32),
            pltpu.SemaphoreType.DMA,
            pltpu.VMEM((_G,), jnp.float32),
            pltpu.VMEM((_G,), jnp.float32),
            pltpu.VMEM_SHARED((_NT * _G,), jnp.float32),
            pltpu.VMEM_SHARED((_NT * _G,), jnp.float32),
            pltpu.VMEM((_NT * _G,), jnp.float32),
            pltpu.VMEM((_NT * _G,), jnp.float32),
            pltpu.VMEM((_G,), jnp.float32),
        ],
    )
    return f(x.reshape(-1), batch)


def kernel(x, edge_index, edge_attr, batch):
    out = _seg_mean(x, batch.astype(jnp.int32))
    return out[:, None]
